# Initial kernel scaffold; baseline (speedup 1.0000x reference)
import jax
import jax.numpy as jnp
from jax.experimental import pallas as pl


def kernel(x, gate_w, w1, w2, w3):
    raise NotImplementedError("write your pallas kernel here")


# speedup vs baseline: 1.5575x; 1.5575x over previous
import functools

import jax
import jax.numpy as jnp
from jax.experimental import pallas as pl
from jax.experimental.pallas import tpu as pltpu

HIDDEN = 1024
FFN = 512
E = 64
T = 128
EPB = 2
_NEG_INF = float("-inf")


def _moe_body(x_ref, gate_ref, w1_ref, w2_ref, w3_ref, out_ref, c_ref):
    i = pl.program_id(0)

    @pl.when(i == 0)
    def _router():
        xb = x_ref[...]
        logits = jax.lax.dot_general(
            xb, gate_ref[...],
            (((1,), (1,)), ((), ())),
            preferred_element_type=jnp.float32,
        )
        iota_e = jax.lax.broadcasted_iota(jnp.int32, (T, E), 1)
        big = jnp.int32(E)
        m1 = jnp.max(logits, axis=1, keepdims=True)
        e1 = jnp.min(jnp.where(logits == m1, iota_e, big), axis=1, keepdims=True)
        masked = jnp.where(iota_e == e1, _NEG_INF, logits)
        m2 = jnp.max(masked, axis=1, keepdims=True)
        e2 = jnp.min(jnp.where(masked == m2, iota_e, big), axis=1, keepdims=True)
        w_top1 = 1.0 / (1.0 + jnp.exp(m2 - m1))
        c_ref[...] = jnp.where(
            iota_e == e1, w_top1,
            jnp.where(iota_e == e2, 1.0 - w_top1, 0.0))
        out_ref[...] = jnp.zeros_like(out_ref)

    xb = x_ref[...]
    h1 = jax.lax.dot_general(
        xb, w1_ref[...], (((1,), (1,)), ((), ())),
        precision=jax.lax.Precision.DEFAULT,
        preferred_element_type=jnp.float32)
    h3 = jax.lax.dot_general(
        xb, w3_ref[...], (((1,), (1,)), ((), ())),
        precision=jax.lax.Precision.DEFAULT,
        preferred_element_type=jnp.float32)
    h = (h1 * jax.lax.logistic(h1)) * h3

    iota_e = jax.lax.broadcasted_iota(jnp.int32, (T, E), 1)
    c = c_ref[...]
    iota_h = jax.lax.broadcasted_iota(jnp.int32, (T, EPB * FFN), 1) // FFN
    scale = jnp.zeros((T, EPB * FFN), jnp.float32)
    for k in range(EPB):
        ck = jnp.sum(jnp.where(iota_e == i * EPB + k, c, 0.0), axis=1,
                     keepdims=True)
        scale = jnp.where(iota_h == k, ck, scale)
    out_ref[...] += jax.lax.dot_general(
        h * scale, w2_ref[...], (((1,), (0,)), ((), ())),
        precision=jax.lax.Precision.DEFAULT,
        preferred_element_type=jnp.float32)


@jax.jit
def kernel(x, gate_w, w1, w2, w3):
    out = pl.pallas_call(
        _moe_body,
        grid=(E // EPB,),
        in_specs=[
            pl.BlockSpec((T, HIDDEN), lambda i: (0, 0)),
            pl.BlockSpec((E, HIDDEN), lambda i: (0, 0)),
            pl.BlockSpec((EPB * FFN, HIDDEN), lambda i: (i, 0)),
            pl.BlockSpec((EPB * FFN, HIDDEN), lambda i: (i, 0)),
            pl.BlockSpec((EPB * FFN, HIDDEN), lambda i: (i, 0)),
        ],
        out_specs=pl.BlockSpec((T, HIDDEN), lambda i: (0, 0)),
        out_shape=jax.ShapeDtypeStruct((T, HIDDEN), jnp.float32),
        scratch_shapes=[pltpu.VMEM((T, E), jnp.float32)],
    )(x, gate_w, w1, w2, w3)
    return out
